# baseline (speedup 1.0000x reference)
import functools

import jax
import jax.numpy as jnp
from jax import lax
from jax.experimental import pallas as pl
from jax.experimental.pallas import tpu as pltpu
from jax.experimental.pallas import tpu_sc as plsc

VOCAB = 100000
EMB = 64
BATCH = 1024
CTX = 20

NC = 2
NS = 16
NW = NC * NS
HALF = EMB // 2
LANES = 16
NBG = BATCH // LANES

_mesh = plsc.VectorSubcoreMesh(core_axis_name="c", subcore_axis_name="s")


@functools.partial(
    pl.kernel,
    out_type=jax.ShapeDtypeStruct((HALF, NBG, LANES), jnp.float32),
    mesh=_mesh,
    scratch_types=[
        pltpu.VMEM((NBG, CTX, LANES), jnp.int32),
        pltpu.VMEM((VOCAB,), jnp.float32),
        pltpu.VMEM((NBG, LANES), jnp.float32),
        pltpu.SemaphoreType.DMA,
    ],
    compiler_params=pltpu.CompilerParams(
        use_tc_tiling_on_sc=False, needs_layout_passes=False
    ),
)
def _sc_pool(ctx_hbm, table_hbm, out_hbm, ctx_v, row_v, acc_v, sem):
    wid = lax.axis_index("s") * NC + lax.axis_index("c")
    row_cp = pltpu.async_copy(table_hbm.at[wid], row_v, sem)
    pltpu.sync_copy(ctx_hbm, ctx_v)
    row_cp.wait()

    def body(bg, carry):
        acc = plsc.load_gather(row_v, [ctx_v[bg, 0, :]])
        for j in range(1, CTX):
            acc = acc + plsc.load_gather(row_v, [ctx_v[bg, j, :]])
        acc_v[bg, :] = acc * (1.0 / CTX)
        return carry

    lax.fori_loop(0, NBG, body, 0)
    pltpu.sync_copy(acc_v, out_hbm.at[wid])


VT = 2048
GRID = (VOCAB + VT - 1) // VT


def _mm_body(wt_ref, b_ref, cv1_ref, cv2_ref, o_ref):
    lhs = jnp.concatenate([wt_ref[...], b_ref[...]], axis=0)
    ones = jnp.ones((1, BATCH), jnp.float32)
    rhs = jnp.concatenate(
        [cv1_ref[...], cv2_ref[...], ones], axis=0
    )
    o_ref[...] = lax.dot_general(
        lhs, rhs,
        (((0,), (0,)), ((), ())),
        preferred_element_type=jnp.float32,
    )


def _project(wt, b2, cv1, cv2):
    return pl.pallas_call(
        _mm_body,
        grid=(GRID,),
        in_specs=[
            pl.BlockSpec((EMB, VT), lambda i: (0, i)),
            pl.BlockSpec((1, VT), lambda i: (0, i)),
            pl.BlockSpec((HALF, BATCH), lambda i: (0, 0)),
            pl.BlockSpec((HALF, BATCH), lambda i: (0, 0)),
        ],
        out_specs=pl.BlockSpec((VT, BATCH), lambda i: (i, 0)),
        out_shape=jax.ShapeDtypeStruct((VOCAB, BATCH), jnp.float32),
    )(wt, b2, cv1, cv2)


def kernel(context, embeddings, W, b):
    ctx_r = context.astype(jnp.int32).reshape(NBG, LANES, CTX).transpose(0, 2, 1)
    emb_t = embeddings.T
    cv1 = _sc_pool(ctx_r, emb_t[:HALF]).reshape(HALF, BATCH)
    cv2 = _sc_pool(ctx_r, emb_t[HALF:]).reshape(HALF, BATCH)
    out_t = _project(W.T, b.reshape(1, VOCAB), cv1, cv2)
    return out_t.T


# speedup vs baseline: 1.0746x; 1.0746x over previous
import functools

import jax
import jax.numpy as jnp
from jax import lax
from jax.experimental import pallas as pl
from jax.experimental.pallas import tpu as pltpu
from jax.experimental.pallas import tpu_sc as plsc

VOCAB = 100000
EMB = 64
BATCH = 1024
CTX = 20

NC = 2
NS = 16
NW = NC * NS
DPW = EMB // NW
LANES = 16
NBG = BATCH // LANES

_mesh = plsc.VectorSubcoreMesh(core_axis_name="c", subcore_axis_name="s")


@functools.partial(
    pl.kernel,
    out_type=jax.ShapeDtypeStruct((EMB, NBG, LANES), jnp.float32),
    mesh=_mesh,
    scratch_types=[
        pltpu.VMEM((NBG, CTX, LANES), jnp.int32),
        pltpu.VMEM((VOCAB,), jnp.float32),
        pltpu.VMEM((DPW, NBG, LANES), jnp.float32),
        pltpu.SemaphoreType.DMA,
    ],
    compiler_params=pltpu.CompilerParams(
        use_tc_tiling_on_sc=False, needs_layout_passes=False
    ),
)
def _sc_pool(ctx_hbm, table_hbm, out_hbm, ctx_v, row_v, acc_v, sem):
    wid = lax.axis_index("s") * NC + lax.axis_index("c")
    pltpu.sync_copy(ctx_hbm, ctx_v)
    for t in range(DPW):
        d = wid * DPW + t
        pltpu.sync_copy(table_hbm.at[d], row_v)

        def body(bg, carry):
            acc = plsc.load_gather(row_v, [ctx_v[bg, 0, :]])
            for j in range(1, CTX):
                acc = acc + plsc.load_gather(row_v, [ctx_v[bg, j, :]])
            acc_v[t, bg, :] = acc * (1.0 / CTX)
            return carry

        lax.fori_loop(0, NBG, body, 0)
    pltpu.sync_copy(acc_v, out_hbm.at[pl.ds(wid * DPW, DPW), :, :])


VT = 4096
GRID = (VOCAB + VT - 1) // VT


def _mm_body(wt_ref, b_ref, cvt_ref, o_ref):
    lhs = jnp.concatenate([wt_ref[...], b_ref[...]], axis=0)
    ones = jnp.ones((1, BATCH), jnp.float32)
    rhs = jnp.concatenate([cvt_ref[...], ones], axis=0)
    o_ref[...] = lax.dot_general(
        lhs, rhs,
        (((0,), (0,)), ((), ())),
        preferred_element_type=jnp.float32,
    )


def _project(wt, b2, cvt):
    return pl.pallas_call(
        _mm_body,
        grid=(GRID,),
        in_specs=[
            pl.BlockSpec((EMB, VT), lambda i: (0, i)),
            pl.BlockSpec((1, VT), lambda i: (0, i)),
            pl.BlockSpec((EMB, BATCH), lambda i: (0, 0)),
        ],
        out_specs=pl.BlockSpec((VT, BATCH), lambda i: (i, 0)),
        out_shape=jax.ShapeDtypeStruct((VOCAB, BATCH), jnp.float32),
    )(wt, b2, cvt)


def kernel(context, embeddings, W, b):
    ctx_r = context.astype(jnp.int32).reshape(NBG, LANES, CTX).transpose(0, 2, 1)
    emb_t = embeddings.T
    cvt = _sc_pool(ctx_r, emb_t).reshape(EMB, BATCH)
    out_t = _project(W.T, b.reshape(1, VOCAB), cvt)
    return out_t.T


# speedup vs baseline: 1.0752x; 1.0005x over previous
import functools

import jax
import jax.numpy as jnp
from jax import lax
from jax.experimental import pallas as pl
from jax.experimental.pallas import tpu as pltpu
from jax.experimental.pallas import tpu_sc as plsc

VOCAB = 100000
EMB = 64
BATCH = 1024
CTX = 20

NC = 2
NS = 16
NW = NC * NS
DPW = EMB // NW
LANES = 16
NBG = BATCH // LANES

_mesh = plsc.VectorSubcoreMesh(core_axis_name="c", subcore_axis_name="s")


@functools.partial(
    pl.kernel,
    out_type=jax.ShapeDtypeStruct((EMB, NBG, LANES), jnp.float32),
    mesh=_mesh,
    scratch_types=[
        pltpu.VMEM((NBG, CTX, LANES), jnp.int32),
        pltpu.VMEM((VOCAB,), jnp.float32),
        pltpu.VMEM((DPW, NBG, LANES), jnp.float32),
        pltpu.SemaphoreType.DMA,
    ],
    compiler_params=pltpu.CompilerParams(
        use_tc_tiling_on_sc=False, needs_layout_passes=False
    ),
)
def _sc_pool(ctx_hbm, table_hbm, out_hbm, ctx_v, row_v, acc_v, sem):
    wid = lax.axis_index("s") * NC + lax.axis_index("c")
    pltpu.sync_copy(ctx_hbm, ctx_v)
    for t in range(DPW):
        d = wid * DPW + t
        pltpu.sync_copy(table_hbm.at[d], row_v)

        def body(bg, carry):
            acc = plsc.load_gather(row_v, [ctx_v[bg, 0, :]])
            for j in range(1, CTX):
                acc = acc + plsc.load_gather(row_v, [ctx_v[bg, j, :]])
            acc_v[t, bg, :] = acc * (1.0 / CTX)
            return carry

        lax.fori_loop(0, NBG, body, 0)
    pltpu.sync_copy(acc_v, out_hbm.at[pl.ds(wid * DPW, DPW), :, :])


VT = 6144
GRID = (VOCAB + VT - 1) // VT


def _mm_body(wt_ref, b_ref, cvt_ref, o_ref):
    lhs = jnp.concatenate([wt_ref[...], b_ref[...]], axis=0)
    ones = jnp.ones((1, BATCH), jnp.float32)
    rhs = jnp.concatenate([cvt_ref[...], ones], axis=0)
    o_ref[...] = lax.dot_general(
        lhs, rhs,
        (((0,), (0,)), ((), ())),
        preferred_element_type=jnp.float32,
    )


def _project(wt, b2, cvt):
    return pl.pallas_call(
        _mm_body,
        grid=(GRID,),
        in_specs=[
            pl.BlockSpec((EMB, VT), lambda i: (0, i)),
            pl.BlockSpec((1, VT), lambda i: (0, i)),
            pl.BlockSpec((EMB, BATCH), lambda i: (0, 0)),
        ],
        out_specs=pl.BlockSpec((VT, BATCH), lambda i: (i, 0)),
        out_shape=jax.ShapeDtypeStruct((VOCAB, BATCH), jnp.float32),
    )(wt, b2, cvt)


def kernel(context, embeddings, W, b):
    ctx_r = context.astype(jnp.int32).reshape(NBG, LANES, CTX).transpose(0, 2, 1)
    emb_t = embeddings.T
    cvt = _sc_pool(ctx_r, emb_t).reshape(EMB, BATCH)
    out_t = _project(W.T, b.reshape(1, VOCAB), cvt)
    return out_t.T


# speedup vs baseline: 1.0756x; 1.0004x over previous
import functools

import jax
import jax.numpy as jnp
from jax import lax
from jax.experimental import pallas as pl
from jax.experimental.pallas import tpu as pltpu
from jax.experimental.pallas import tpu_sc as plsc

VOCAB = 100000
EMB = 64
BATCH = 1024
CTX = 20

NC = 2
NS = 16
NW = NC * NS
DPW = EMB // NW
LANES = 16
NBG = BATCH // LANES

_mesh = plsc.VectorSubcoreMesh(core_axis_name="c", subcore_axis_name="s")


@functools.partial(
    pl.kernel,
    out_type=jax.ShapeDtypeStruct((EMB, NBG, LANES), jnp.float32),
    mesh=_mesh,
    scratch_types=[
        pltpu.VMEM((NBG, CTX, LANES), jnp.int32),
        pltpu.VMEM((VOCAB,), jnp.float32),
        pltpu.VMEM((DPW, NBG, LANES), jnp.float32),
        pltpu.SemaphoreType.DMA,
    ],
    compiler_params=pltpu.CompilerParams(
        use_tc_tiling_on_sc=False, needs_layout_passes=False
    ),
)
def _sc_pool(ctx_hbm, table_hbm, out_hbm, ctx_v, row_v, acc_v, sem):
    wid = lax.axis_index("s") * NC + lax.axis_index("c")
    pltpu.sync_copy(ctx_hbm, ctx_v)
    for t in range(DPW):
        d = wid * DPW + t
        pltpu.sync_copy(table_hbm.at[d], row_v)

        def body(bg, carry):
            acc = plsc.load_gather(row_v, [ctx_v[bg, 0, :]])
            for j in range(1, CTX):
                acc = acc + plsc.load_gather(row_v, [ctx_v[bg, j, :]])
            acc_v[t, bg, :] = acc * (1.0 / CTX)
            return carry

        lax.fori_loop(0, NBG, body, 0)
    pltpu.sync_copy(acc_v, out_hbm.at[pl.ds(wid * DPW, DPW), :, :])


VT = 4096
GRID = (VOCAB + VT - 1) // VT


def _mm_body(wt_ref, b_ref, cvt_ref, o_ref):
    lhs = jnp.concatenate([wt_ref[...], b_ref[...]], axis=0)
    ones = jnp.ones((1, BATCH), jnp.float32)
    rhs = jnp.concatenate([cvt_ref[...], ones], axis=0)
    o_ref[...] = lax.dot_general(
        lhs, rhs,
        (((0,), (0,)), ((), ())),
        preferred_element_type=jnp.float32,
    )


def _project(wt, b2, cvt):
    return pl.pallas_call(
        _mm_body,
        grid=(GRID,),
        in_specs=[
            pl.BlockSpec((EMB, VT), lambda i: (0, i)),
            pl.BlockSpec((1, VT), lambda i: (0, i)),
            pl.BlockSpec((EMB, BATCH), lambda i: (0, 0)),
        ],
        out_specs=pl.BlockSpec((VT, BATCH), lambda i: (i, 0)),
        out_shape=jax.ShapeDtypeStruct((VOCAB, BATCH), jnp.float32),
    )(wt, b2, cvt)


def kernel(context, embeddings, W, b):
    ctx_r = context.astype(jnp.int32).reshape(NBG, LANES, CTX).transpose(0, 2, 1)
    emb_t = embeddings.T
    cvt = _sc_pool(ctx_r, emb_t).reshape(EMB, BATCH)
    out_t = _project(W.T, b.reshape(1, VOCAB), cvt)
    return out_t.T
